# baseline (speedup 1.0000x reference)
import functools

import jax
import jax.numpy as jnp
from jax import lax
from jax.experimental import pallas as pl
from jax.experimental.pallas import tpu as pltpu
from jax.experimental.pallas import tpu_sc as plsc

BATCH = 4096
EMB_DIM = 128
MAX_NORM = 1.0
EPS = 1e-7

NUM_CORES = 2
NUM_SUBCORES = 16
LANES = 16
NUM_WORKERS = NUM_CORES * NUM_SUBCORES
B_PER_W = BATCH // NUM_WORKERS
CHUNKS = EMB_DIM // LANES
GROUP = LANES
N_GROUPS = B_PER_W // GROUP


def _sc_lookup(idx, table):
  mesh = plsc.VectorSubcoreMesh(core_axis_name="c", subcore_axis_name="s")

  @functools.partial(
      pl.kernel,
      mesh=mesh,
      out_type=jax.ShapeDtypeStruct((BATCH, EMB_DIM), jnp.float32),
      scratch_types=[
          pltpu.VMEM((B_PER_W,), jnp.int32),
          pltpu.VMEM((B_PER_W, EMB_DIM), jnp.float32),
          pltpu.VMEM((B_PER_W, EMB_DIM), jnp.float32),
          pltpu.VMEM((GROUP, LANES), jnp.float32),
          [pltpu.SemaphoreType.DMA] * N_GROUPS,
          [pltpu.SemaphoreType.DMA] * N_GROUPS,
      ],
      compiler_params=pltpu.CompilerParams(needs_layout_passes=False),
  )
  def body(idx_hbm, table_hbm, out_hbm, idx_v, rows_v, out_v, ss_mat,
           gsems, wsems):
    wid = lax.axis_index("s") * NUM_CORES + lax.axis_index("c")
    base = wid * B_PER_W
    pltpu.sync_copy(idx_hbm.at[pl.ds(base, B_PER_W)], idx_v)
    gathers = []
    for g in range(N_GROUPS):
      r0 = g * GROUP
      gathers.append(
          pltpu.async_copy(table_hbm.at[idx_v.at[pl.ds(r0, GROUP)]],
                           rows_v.at[pl.ds(r0, GROUP)], gsems[g]))

    iota = lax.iota(jnp.int32, LANES)
    writes = []
    for g in range(N_GROUPS):
      r0 = g * GROUP
      gathers[g].wait()
      for j in range(GROUP):
        acc = jnp.zeros((LANES,), jnp.float32)
        for c in range(CHUNKS):
          v = rows_v[r0 + j, pl.ds(c * LANES, LANES)]
          acc = acc + v * v
        ss_mat[j, :] = acc
      tot = jnp.zeros((LANES,), jnp.float32)
      for l in range(LANES):
        col = plsc.load_gather(ss_mat, [iota, jnp.full((LANES,), l, jnp.int32)])
        tot = tot + col
      tot = jnp.maximum(tot, 1e-30)
      i = lax.bitcast_convert_type(tot, jnp.int32)
      i = 0x5F3759DF - lax.shift_right_logical(i, 1)
      y = lax.bitcast_convert_type(i, jnp.float32)
      half = 0.5 * tot
      for _ in range(3):
        y = y * (1.5 - half * y * y)
      norm = tot * y
      scale = jnp.where(norm > MAX_NORM, MAX_NORM / (norm + EPS),
                        jnp.float32(1.0))
      for j in range(GROUP):
        splat = jnp.full((LANES,), scale[j], jnp.float32)
        for c in range(CHUNKS):
          out_v[r0 + j, pl.ds(c * LANES, LANES)] = (
              rows_v[r0 + j, pl.ds(c * LANES, LANES)] * splat)
      writes.append(
          pltpu.async_copy(out_v.at[pl.ds(r0, GROUP)],
                           out_hbm.at[pl.ds(base + r0, GROUP)], wsems[g]))
    for w in writes:
      w.wait()

  return body(idx, table)


def kernel(batch_radicalindices, rademb_weight):
  idx = batch_radicalindices.reshape(-1).astype(jnp.int32)
  out = _sc_lookup(idx, rademb_weight)
  return out.reshape(BATCH, 1, EMB_DIM)


# speedup vs baseline: 1.0262x; 1.0262x over previous
import functools

import jax
import jax.numpy as jnp
from jax import lax
from jax.experimental import pallas as pl
from jax.experimental.pallas import tpu as pltpu
from jax.experimental.pallas import tpu_sc as plsc

BATCH = 4096
EMB_DIM = 128
MAX_NORM = 1.0
EPS = 1e-7

NUM_CORES = 2
NUM_SUBCORES = 16
LANES = 16
NUM_WORKERS = NUM_CORES * NUM_SUBCORES
B_PER_W = BATCH // NUM_WORKERS
CHUNKS = EMB_DIM // LANES
GROUP = LANES
N_GROUPS = B_PER_W // GROUP
HALF = B_PER_W // 2


def _sc_lookup(idx, table):
  mesh = plsc.VectorSubcoreMesh(core_axis_name="c", subcore_axis_name="s")

  @functools.partial(
      pl.kernel,
      mesh=mesh,
      out_type=jax.ShapeDtypeStruct((BATCH, EMB_DIM), jnp.float32),
      scratch_types=[
          pltpu.VMEM((B_PER_W,), jnp.int32),
          pltpu.VMEM((B_PER_W, EMB_DIM), jnp.float32),
          pltpu.VMEM((B_PER_W, EMB_DIM), jnp.float32),
          pltpu.VMEM((GROUP, LANES), jnp.float32),
          pltpu.VMEM((GROUP,), jnp.float32),
          pltpu.SemaphoreType.DMA,
          pltpu.SemaphoreType.DMA,
          pltpu.SemaphoreType.DMA,
      ],
      compiler_params=pltpu.CompilerParams(needs_layout_passes=False),
  )
  def body(idx_hbm, table_hbm, out_hbm, idx_v, rows_v, out_v, ss_mat,
           scale_v, sem0, sem1, wsem):
    wid = lax.axis_index("s") * NUM_CORES + lax.axis_index("c")
    base = wid * B_PER_W
    pltpu.sync_copy(idx_hbm.at[pl.ds(base, B_PER_W)], idx_v)
    g0 = pltpu.async_copy(table_hbm.at[idx_v.at[pl.ds(0, HALF)]],
                          rows_v.at[pl.ds(0, HALF)], sem0)
    g1 = pltpu.async_copy(table_hbm.at[idx_v.at[pl.ds(HALF, HALF)]],
                          rows_v.at[pl.ds(HALF, HALF)], sem1)

    iota = lax.iota(jnp.int32, LANES)

    def group_fn(g, carry):
      r0 = g * GROUP

      @pl.when(g == 0)
      def _():
        g0.wait()

      @pl.when(g == N_GROUPS // 2)
      def _():
        g1.wait()

      def p1(j, c1):
        acc = jnp.zeros((LANES,), jnp.float32)
        for c in range(CHUNKS):
          v = rows_v[r0 + j, pl.ds(c * LANES, LANES)]
          acc = acc + v * v
        ss_mat[j, :] = acc
        return c1

      lax.fori_loop(0, GROUP, p1, 0)
      tot = jnp.zeros((LANES,), jnp.float32)
      for l in range(LANES):
        col = plsc.load_gather(ss_mat, [iota, jnp.full((LANES,), l, jnp.int32)])
        tot = tot + col
      tot = jnp.maximum(tot, 1e-30)
      i = lax.bitcast_convert_type(tot, jnp.int32)
      i = 0x5F3759DF - lax.shift_right_logical(i, 1)
      y = lax.bitcast_convert_type(i, jnp.float32)
      half = 0.5 * tot
      for _ in range(3):
        y = y * (1.5 - half * y * y)
      norm = tot * y
      scale = jnp.where(norm > MAX_NORM, MAX_NORM / (norm + EPS),
                        jnp.float32(1.0))
      scale_v[:] = scale

      def p2(j, c2):
        splat = plsc.load_gather(scale_v, [jnp.full((LANES,), 0, jnp.int32) + j])
        for c in range(CHUNKS):
          out_v[r0 + j, pl.ds(c * LANES, LANES)] = (
              rows_v[r0 + j, pl.ds(c * LANES, LANES)] * splat)
        return c2

      lax.fori_loop(0, GROUP, p2, 0)
      pltpu.async_copy(out_v.at[pl.ds(r0, GROUP)],
                       out_hbm.at[pl.ds(base + r0, GROUP)], wsem)
      return carry

    lax.fori_loop(0, N_GROUPS, group_fn, 0)
    pltpu.make_async_copy(out_v, out_hbm.at[pl.ds(base, B_PER_W)],
                          wsem).wait()

  return body(idx, table)


def kernel(batch_radicalindices, rademb_weight):
  idx = batch_radicalindices.reshape(-1).astype(jnp.int32)
  out = _sc_lookup(idx, rademb_weight)
  return out.reshape(BATCH, 1, EMB_DIM)


# speedup vs baseline: 1.1854x; 1.1551x over previous
import functools

import jax
import jax.numpy as jnp
from jax import lax
from jax.experimental import pallas as pl
from jax.experimental.pallas import tpu as pltpu
from jax.experimental.pallas import tpu_sc as plsc

BATCH = 4096
EMB_DIM = 128
MAX_NORM = 1.0
EPS = 1e-7

NUM_CORES = 2
NUM_SUBCORES = 16
LANES = 16
NUM_WORKERS = NUM_CORES * NUM_SUBCORES
B_PER_W = BATCH // NUM_WORKERS
CHUNKS = EMB_DIM // LANES
GROUP = LANES
N_GROUPS = B_PER_W // GROUP


def _tree_sum(vals):
  vals = list(vals)
  while len(vals) > 1:
    nxt = [a + b for a, b in zip(vals[::2], vals[1::2])]
    if len(vals) % 2:
      nxt.append(vals[-1])
    vals = nxt
  return vals[0]


def _sc_lookup(idx, table):
  mesh = plsc.VectorSubcoreMesh(core_axis_name="c", subcore_axis_name="s")

  @functools.partial(
      pl.kernel,
      mesh=mesh,
      out_type=jax.ShapeDtypeStruct((BATCH, EMB_DIM), jnp.float32),
      scratch_types=[
          pltpu.VMEM((B_PER_W,), jnp.int32),
          pltpu.VMEM((B_PER_W, EMB_DIM), jnp.float32),
          pltpu.VMEM((B_PER_W, EMB_DIM), jnp.float32),
          pltpu.VMEM((GROUP, LANES), jnp.float32),
          [pltpu.SemaphoreType.DMA] * N_GROUPS,
          pltpu.SemaphoreType.DMA,
      ],
      compiler_params=pltpu.CompilerParams(needs_layout_passes=False),
  )
  def body(idx_hbm, table_hbm, out_hbm, idx_v, rows_v, out_v, ss_mat,
           gsems, wsem):
    wid = lax.axis_index("s") * NUM_CORES + lax.axis_index("c")
    base = wid * B_PER_W
    pltpu.sync_copy(idx_hbm.at[pl.ds(base, B_PER_W)], idx_v)
    gathers = []
    for g in range(N_GROUPS):
      r0 = g * GROUP
      gathers.append(
          pltpu.async_copy(table_hbm.at[idx_v.at[pl.ds(r0, GROUP)]],
                           rows_v.at[pl.ds(r0, GROUP)], gsems[g]))

    iota = lax.iota(jnp.int32, LANES)

    def group_fn(g, carry):
      r0 = g * GROUP
      for k in range(N_GROUPS):
        @pl.when(g == k)
        def _(k=k):
          gathers[k].wait()

      for j in range(GROUP):
        vs = [rows_v[r0 + j, pl.ds(c * LANES, LANES)] for c in range(CHUNKS)]
        ss_mat[j, :] = _tree_sum([v * v for v in vs])
      cols = [
          plsc.load_gather(ss_mat, [iota, jnp.full((LANES,), l, jnp.int32)])
          for l in range(LANES)
      ]
      tot = jnp.maximum(_tree_sum(cols), 1e-30)
      i = lax.bitcast_convert_type(tot, jnp.int32)
      i = 0x5F3759DF - lax.shift_right_logical(i, 1)
      y = lax.bitcast_convert_type(i, jnp.float32)
      half = 0.5 * tot
      for _ in range(3):
        y = y * (1.5 - half * y * y)
      norm = tot * y
      scale = jnp.where(norm > MAX_NORM, MAX_NORM / (norm + EPS),
                        jnp.float32(1.0))
      for j in range(GROUP):
        splat = jnp.full((LANES,), scale[j], jnp.float32)
        for c in range(CHUNKS):
          out_v[r0 + j, pl.ds(c * LANES, LANES)] = (
              rows_v[r0 + j, pl.ds(c * LANES, LANES)] * splat)
      pltpu.async_copy(out_v.at[pl.ds(r0, GROUP)],
                       out_hbm.at[pl.ds(base + r0, GROUP)], wsem)
      return carry

    lax.fori_loop(0, N_GROUPS, group_fn, 0)
    pltpu.make_async_copy(out_v, out_hbm.at[pl.ds(base, B_PER_W)],
                          wsem).wait()

  return body(idx, table)


def kernel(batch_radicalindices, rademb_weight):
  idx = batch_radicalindices.reshape(-1).astype(jnp.int32)
  out = _sc_lookup(idx, rademb_weight)
  return out.reshape(BATCH, 1, EMB_DIM)


# speedup vs baseline: 1.3113x; 1.1062x over previous
import functools

import jax
import jax.numpy as jnp
from jax import lax
from jax.experimental import pallas as pl
from jax.experimental.pallas import tpu as pltpu
from jax.experimental.pallas import tpu_sc as plsc

BATCH = 4096
EMB_DIM = 128
MAX_NORM = 1.0

NUM_CORES = 2
NUM_SUBCORES = 16
LANES = 16
NUM_WORKERS = NUM_CORES * NUM_SUBCORES
B_PER_W = BATCH // NUM_WORKERS
CHUNKS = EMB_DIM // LANES
GROUP = LANES
N_GROUPS = B_PER_W // GROUP


def _tree_sum(vals):
  vals = list(vals)
  while len(vals) > 1:
    nxt = [a + b for a, b in zip(vals[::2], vals[1::2])]
    if len(vals) % 2:
      nxt.append(vals[-1])
    vals = nxt
  return vals[0]


def _sc_lookup(idx, table):
  mesh = plsc.VectorSubcoreMesh(core_axis_name="c", subcore_axis_name="s")

  @functools.partial(
      pl.kernel,
      mesh=mesh,
      out_type=jax.ShapeDtypeStruct((BATCH, EMB_DIM), jnp.float32),
      scratch_types=[
          pltpu.VMEM((B_PER_W,), jnp.int32),
          pltpu.VMEM((B_PER_W, EMB_DIM), jnp.float32),
          pltpu.VMEM((B_PER_W, EMB_DIM), jnp.float32),
          [pltpu.SemaphoreType.DMA] * N_GROUPS,
          pltpu.SemaphoreType.DMA,
      ],
      compiler_params=pltpu.CompilerParams(needs_layout_passes=False),
  )
  def body(idx_hbm, table_hbm, out_hbm, idx_v, rows_v, out_v, gsems, wsem):
    wid = lax.axis_index("s") * NUM_CORES + lax.axis_index("c")
    base = wid * B_PER_W
    pltpu.sync_copy(idx_hbm.at[pl.ds(base, B_PER_W)], idx_v)
    gathers = []
    for g in range(N_GROUPS):
      r0 = g * GROUP
      gathers.append(
          pltpu.async_copy(table_hbm.at[idx_v.at[pl.ds(r0, GROUP)]],
                           rows_v.at[pl.ds(r0, GROUP)], gsems[g]))

    def group_fn(g, carry):
      r0 = g * GROUP
      for k in range(N_GROUPS):
        @pl.when(g == k)
        def _(k=k):
          gathers[k].wait()

      for j in range(GROUP):
        vs = [rows_v[r0 + j, pl.ds(c * LANES, LANES)] for c in range(CHUNKS)]
        sq = _tree_sum([v * v for v in vs])
        s = lax.reduce_sum_p.bind(sq, axes=(0,))
        s = jnp.maximum(s, jnp.float32(1e-30))
        i = lax.bitcast_convert_type(s, jnp.int32)
        i = 0x5F3759DF - lax.shift_right_logical(i, 1)
        y = lax.bitcast_convert_type(i, jnp.float32)
        half = 0.5 * s
        for _ in range(3):
          y = y * (1.5 - half * y * y)
        sc = jnp.where(s * y > MAX_NORM, y, jnp.float32(1.0))
        splat = jnp.full((LANES,), sc, jnp.float32)
        for c in range(CHUNKS):
          out_v[r0 + j, pl.ds(c * LANES, LANES)] = vs[c] * splat
      pltpu.async_copy(out_v.at[pl.ds(r0, GROUP)],
                       out_hbm.at[pl.ds(base + r0, GROUP)], wsem)
      return carry

    lax.fori_loop(0, N_GROUPS, group_fn, 0)
    pltpu.make_async_copy(out_v, out_hbm.at[pl.ds(base, B_PER_W)],
                          wsem).wait()

  return body(idx, table)


def kernel(batch_radicalindices, rademb_weight):
  idx = batch_radicalindices.reshape(-1).astype(jnp.int32)
  out = _sc_lookup(idx, rademb_weight)
  return out.reshape(BATCH, 1, EMB_DIM)
